# Initial kernel scaffold; baseline (speedup 1.0000x reference)
import jax
import jax.numpy as jnp
from jax.experimental import pallas as pl


def kernel(row_sparse_feat, w):
    raise NotImplementedError("write your pallas kernel here")


# speedup vs baseline: 1.3253x; 1.3253x over previous
import functools

import jax
import jax.numpy as jnp
from jax import lax
from jax.experimental import pallas as pl
from jax.experimental.pallas import tpu as pltpu
from jax.experimental.pallas import tpu_sc as plsc

FEAT_LEN = 1000000
BATCH = 16384
N_FIELDS = 26

NC = 2
NS = 16
L = 16
NW = NC * NS
BPW = BATCH // NW
NIDX = BPW * N_FIELDS
CH = 128
NCH = NIDX // CH

_mesh = plsc.VectorSubcoreMesh(core_axis_name="c", subcore_axis_name="s")


@functools.partial(
    pl.kernel,
    out_type=jax.ShapeDtypeStruct((BATCH,), jnp.float32),
    mesh=_mesh,
    compiler_params=pltpu.CompilerParams(needs_layout_passes=False),
    scratch_types=[
        pltpu.VMEM((NCH, CH), jnp.int32),
        pltpu.VMEM((NIDX,), jnp.float32),
        pltpu.VMEM((BPW,), jnp.float32),
        pltpu.SemaphoreType.DMA,
    ],
)
def _lookup_sum(idx_hbm, tab_hbm, out_hbm, idx_v, vals_v, acc_v, sem):
    wid = lax.axis_index("s") * NC + lax.axis_index("c")
    row0 = wid * NCH

    pltpu.sync_copy(idx_hbm.at[pl.ds(row0, NCH)], idx_v)

    def fire(j, _):
        pltpu.async_copy(tab_hbm.at[idx_v.at[j]],
                         vals_v.at[pl.ds(j * CH, CH)], sem)
        return _

    lax.fori_loop(0, NCH, fire, 0)

    def drain(j, _):
        pltpu.make_async_copy(tab_hbm.at[idx_v.at[j]],
                              vals_v.at[pl.ds(j * CH, CH)], sem).wait()
        return _

    lax.fori_loop(0, NCH, drain, 0)

    iota = lax.iota(jnp.int32, L)

    def reduce_chunk(j, _):
        base = (j * L + iota) * N_FIELDS
        acc = plsc.load_gather(vals_v, [base])
        for f in range(1, N_FIELDS):
            acc = acc + plsc.load_gather(vals_v, [base + f])
        acc_v[pl.ds(j * L, L)] = acc
        return _

    lax.fori_loop(0, BPW // L, reduce_chunk, 0)

    pltpu.sync_copy(acc_v, out_hbm.at[pl.ds(wid * BPW, BPW)])


def kernel(row_sparse_feat, w):
    idx2d = row_sparse_feat.reshape(BATCH * N_FIELDS // CH, CH)
    tab = w.reshape(FEAT_LEN)
    out = _lookup_sum(idx2d, tab)
    return out.reshape(BATCH, 1)


# speedup vs baseline: 2.7775x; 2.0957x over previous
import functools

import jax
import jax.numpy as jnp
from jax import lax
from jax.experimental import pallas as pl
from jax.experimental.pallas import tpu as pltpu
from jax.experimental.pallas import tpu_sc as plsc

FEAT_LEN = 1000000
BATCH = 16384
N_FIELDS = 26

NC = 2
NS = 16
L = 16
NW = NC * NS
BPW = BATCH // NW
NIDX = BPW * N_FIELDS
CH = 128
NCH = NIDX // CH

NF_PAD = 32
TAB_PAD = 1000448

_mesh = plsc.VectorSubcoreMesh(core_axis_name="c", subcore_axis_name="s")


@functools.partial(
    pl.kernel,
    out_type=jax.ShapeDtypeStruct((BATCH,), jnp.float32),
    mesh=_mesh,
    compiler_params=pltpu.CompilerParams(needs_layout_passes=False),
    scratch_types=[
        pltpu.VMEM((4, 4, 8, CH), jnp.int32),
        pltpu.VMEM((NIDX,), jnp.float32),
        pltpu.VMEM((BPW,), jnp.float32),
        pltpu.SemaphoreType.DMA,
    ],
)
def _lookup_sum(idx_hbm, tab_hbm, out_hbm, idx_v, vals_v, acc_v, sem):
    wid = lax.axis_index("s") * NC + lax.axis_index("c")

    pltpu.sync_copy(idx_hbm.at[:, pl.ds(wid * 4, 4)], idx_v)

    def fire(j, _):
        f = j // 4
        c = j - f * 4
        pltpu.async_copy(tab_hbm.at[idx_v.at[f // 8, c, f % 8]],
                         vals_v.at[pl.ds(j * CH, CH)], sem)
        return _

    lax.fori_loop(0, NCH, fire, 0)

    def drain(j, _):
        f = j // 4
        c = j - f * 4
        pltpu.make_async_copy(tab_hbm.at[idx_v.at[f // 8, c, f % 8]],
                              vals_v.at[pl.ds(j * CH, CH)], sem).wait()
        return _

    lax.fori_loop(0, NCH, drain, 0)

    def reduce_chunk(j, _):
        col = j * L
        acc = vals_v[pl.ds(col, L)]
        for f in range(1, N_FIELDS):
            acc = acc + vals_v[pl.ds(f * BPW + col, L)]
        acc_v[pl.ds(col, L)] = acc
        return _

    lax.fori_loop(0, BPW // L, reduce_chunk, 0)

    pltpu.sync_copy(acc_v, out_hbm.at[pl.ds(wid * BPW, BPW)])


def kernel(row_sparse_feat, w):
    idx4 = (jnp.pad(row_sparse_feat, ((0, 0), (0, NF_PAD - N_FIELDS)))
            .reshape(BATCH // CH, CH, NF_PAD // 8, 8)
            .transpose(2, 0, 3, 1))
    tab = jnp.pad(w, ((0, TAB_PAD - FEAT_LEN), (0, 0))).reshape(TAB_PAD)
    out = _lookup_sum(idx4, tab)
    return out.reshape(BATCH, 1)


# speedup vs baseline: 3.0220x; 1.0881x over previous
import functools

import jax
import jax.numpy as jnp
from jax import lax
from jax.experimental import pallas as pl
from jax.experimental.pallas import tpu as pltpu
from jax.experimental.pallas import tpu_sc as plsc

FEAT_LEN = 1000000
BATCH = 16384
N_FIELDS = 26

NC = 2
NS = 16
L = 16
NW = NC * NS
BPW = BATCH // NW
NIDX = BPW * N_FIELDS
CH = 128
NCH = NIDX // CH

NF_PAD = 32
TAB_PAD = 1000448

_mesh = plsc.VectorSubcoreMesh(core_axis_name="c", subcore_axis_name="s")


@functools.partial(
    pl.kernel,
    out_type=jax.ShapeDtypeStruct((BATCH,), jnp.float32),
    mesh=_mesh,
    compiler_params=pltpu.CompilerParams(needs_layout_passes=False),
    scratch_types=[
        pltpu.VMEM((4, 4, 8, CH), jnp.int32),
        pltpu.VMEM((4, 4, 8, CH), jnp.float32),
        pltpu.VMEM((BPW,), jnp.float32),
        pltpu.VMEM_SHARED((TAB_PAD,), jnp.float32),
        pltpu.VMEM((TAB_PAD // (NS * 8),), jnp.float32),
        pltpu.VMEM((TAB_PAD // (NS * 8),), jnp.float32),
        pltpu.SemaphoreType.DMA,
        pltpu.SemaphoreType.DMA,
    ],
)
def _lookup_sum(idx_hbm, tab_hbm, out_hbm, idx_v, vals_v, acc_v, tab_s,
                stage_a, stage_b, sem, sem2):
    cid = lax.axis_index("c")
    sid = lax.axis_index("s")
    wid = sid * NC + cid

    pltpu.sync_copy(idx_hbm.at[:, pl.ds(wid * 4, 4)], idx_v)

    part = TAB_PAD // NS
    cht = part // 8
    base = sid * part
    bufs = (stage_a, stage_b)
    pltpu.async_copy(tab_hbm.at[pl.ds(base, cht)], bufs[0], sem2)
    for k in range(8):
        b = bufs[k & 1]
        pltpu.make_async_copy(tab_hbm.at[pl.ds(base + k * cht, cht)],
                              b, sem2).wait()
        if k < 7:
            pltpu.async_copy(tab_hbm.at[pl.ds(base + (k + 1) * cht, cht)],
                             bufs[(k + 1) & 1], sem2)
        pltpu.sync_copy(b, tab_s.at[pl.ds(base + k * cht, cht)])
    plsc.subcore_barrier()

    def fire(j, _):
        f = j // 4
        c = j - f * 4
        pltpu.async_copy(tab_s.at[idx_v.at[f // 8, c, f % 8]],
                         vals_v.at[f // 8, c, f % 8], sem)
        return _

    lax.fori_loop(0, NCH, fire, 0)

    def drain(j, _):
        f = j // 4
        c = j - f * 4
        pltpu.make_async_copy(tab_s.at[idx_v.at[f // 8, c, f % 8]],
                              vals_v.at[f // 8, c, f % 8], sem).wait()
        return _

    lax.fori_loop(0, NCH, drain, 0)

    def reduce_chunk(j, _):
        c = j // 8
        l = j - c * 8
        col = l * L
        acc = vals_v[0, c, 0, pl.ds(col, L)]
        for f in range(1, N_FIELDS):
            acc = acc + vals_v[f // 8, c, f % 8, pl.ds(col, L)]
        acc_v[pl.ds(c * CH + col, L)] = acc
        return _

    lax.fori_loop(0, BPW // L, reduce_chunk, 0)

    pltpu.sync_copy(acc_v, out_hbm.at[pl.ds(wid * BPW, BPW)])


def kernel(row_sparse_feat, w):
    idx4 = (jnp.pad(row_sparse_feat, ((0, 0), (0, NF_PAD - N_FIELDS)))
            .reshape(BATCH // CH, CH, NF_PAD // 8, 8)
            .transpose(2, 0, 3, 1))
    tab = jnp.pad(w, ((0, TAB_PAD - FEAT_LEN), (0, 0))).reshape(TAB_PAD)
    out = _lookup_sum(idx4, tab)
    return out.reshape(BATCH, 1)


# speedup vs baseline: 3.1826x; 1.0531x over previous
import functools

import jax
import jax.numpy as jnp
from jax import lax
from jax.experimental import pallas as pl
from jax.experimental.pallas import tpu as pltpu
from jax.experimental.pallas import tpu_sc as plsc

FEAT_LEN = 1000000
BATCH = 16384
N_FIELDS = 26

NC = 2
NS = 16
L = 16
NW = NC * NS
BPW = BATCH // NW
CH = 128
NCH = BPW * N_FIELDS // CH

TAB_MAIN = 999424
TAB_TAIL = 1024
TAB_PAD = TAB_MAIN + TAB_TAIL
PART = TAB_MAIN // NS
CHT = PART // 8

_mesh = plsc.VectorSubcoreMesh(core_axis_name="c", subcore_axis_name="s")


@functools.partial(
    pl.kernel,
    out_type=jax.ShapeDtypeStruct((BATCH,), jnp.float32),
    mesh=_mesh,
    compiler_params=pltpu.CompilerParams(needs_layout_passes=False),
    scratch_types=[
        pltpu.VMEM((N_FIELDS, BPW), jnp.int32),
        pltpu.VMEM((N_FIELDS, BPW), jnp.float32),
        pltpu.VMEM((BPW,), jnp.float32),
        pltpu.VMEM_SHARED((TAB_PAD,), jnp.float32),
        pltpu.VMEM((CHT,), jnp.float32),
        pltpu.VMEM((CHT,), jnp.float32),
        pltpu.SemaphoreType.DMA,
        pltpu.SemaphoreType.DMA,
    ],
)
def _lookup_sum(idx_hbm, tabm_hbm, tabt_hbm, out_hbm, idx_v, vals_v, acc_v,
                tab_s, stage_a, stage_b, sem, sem2):
    cid = lax.axis_index("c")
    sid = lax.axis_index("s")
    wid = sid * NC + cid

    pltpu.sync_copy(idx_hbm.at[:, pl.ds(wid * BPW, BPW)], idx_v)

    base = sid * PART
    bufs = (stage_a, stage_b)
    pltpu.async_copy(tabm_hbm.at[pl.ds(base, CHT)], bufs[0], sem2)
    for k in range(8):
        b = bufs[k & 1]
        pltpu.make_async_copy(tabm_hbm.at[pl.ds(base + k * CHT, CHT)],
                              b, sem2).wait()
        if k < 7:
            pltpu.async_copy(tabm_hbm.at[pl.ds(base + (k + 1) * CHT, CHT)],
                             bufs[(k + 1) & 1], sem2)
        pltpu.sync_copy(b, tab_s.at[pl.ds(base + k * CHT, CHT)])

    @pl.when(sid == NS - 1)
    def _tail():
        pltpu.sync_copy(tabt_hbm, stage_a.at[pl.ds(0, TAB_TAIL)])
        pltpu.sync_copy(stage_a.at[pl.ds(0, TAB_TAIL)],
                        tab_s.at[pl.ds(TAB_MAIN, TAB_TAIL)])

    plsc.subcore_barrier()

    def fire(j, _):
        f = j // 4
        c = j - f * 4
        pltpu.async_copy(tab_s.at[idx_v.at[f, pl.ds(c * CH, CH)]],
                         vals_v.at[f, pl.ds(c * CH, CH)], sem)
        return _

    lax.fori_loop(0, NCH, fire, 0)

    def drain(j, _):
        f = j // 4
        c = j - f * 4
        pltpu.make_async_copy(tab_s.at[idx_v.at[f, pl.ds(c * CH, CH)]],
                              vals_v.at[f, pl.ds(c * CH, CH)], sem).wait()
        return _

    lax.fori_loop(0, NCH, drain, 0)

    def reduce_chunk(j, _):
        col = j * L
        acc = vals_v[0, pl.ds(col, L)]
        for f in range(1, N_FIELDS):
            acc = acc + vals_v[f, pl.ds(col, L)]
        acc_v[pl.ds(col, L)] = acc
        return _

    lax.fori_loop(0, BPW // L, reduce_chunk, 0)

    pltpu.sync_copy(acc_v, out_hbm.at[pl.ds(wid * BPW, BPW)])


def kernel(row_sparse_feat, w):
    idx_t = row_sparse_feat.T
    tab_main = w[:TAB_MAIN].reshape(TAB_MAIN)
    tab_tail = jnp.pad(w[TAB_MAIN:],
                       ((0, TAB_TAIL - (FEAT_LEN - TAB_MAIN)), (0, 0))
                       ).reshape(TAB_TAIL)
    out = _lookup_sum(idx_t, tab_main, tab_tail)
    return out.reshape(BATCH, 1)


# speedup vs baseline: 3.2654x; 1.0260x over previous
import functools

import jax
import jax.numpy as jnp
from jax import lax
from jax.experimental import pallas as pl
from jax.experimental.pallas import tpu as pltpu
from jax.experimental.pallas import tpu_sc as plsc

FEAT_LEN = 1000000
BATCH = 16384
N_FIELDS = 26

NC = 2
NS = 16
L = 16
NW = NC * NS
BPW = BATCH // NW
CH = 128
NCH = BPW * N_FIELDS // CH

TAB_MAIN = 999424
TAB_TAIL = 1024
TAB_PAD = TAB_MAIN + TAB_TAIL
PART = TAB_MAIN // NS
CHT = PART // 8

_mesh = plsc.VectorSubcoreMesh(core_axis_name="c", subcore_axis_name="s")


@functools.partial(
    pl.kernel,
    out_type=jax.ShapeDtypeStruct((BATCH,), jnp.float32),
    mesh=_mesh,
    compiler_params=pltpu.CompilerParams(needs_layout_passes=False),
    scratch_types=[
        pltpu.VMEM((N_FIELDS * BPW,), jnp.int32),
        pltpu.VMEM((N_FIELDS * BPW,), jnp.float32),
        pltpu.VMEM((BPW,), jnp.float32),
        pltpu.VMEM_SHARED((TAB_PAD,), jnp.float32),
        pltpu.VMEM((CHT,), jnp.float32),
        pltpu.VMEM((CHT,), jnp.float32),
        pltpu.SemaphoreType.DMA,
        pltpu.SemaphoreType.DMA,
    ],
)
def _lookup_sum(idx_hbm, tabm_hbm, tabt_hbm, out_hbm, idx_v, vals_v, acc_v,
                tab_s, stage_a, stage_b, sem, sem2):
    cid = lax.axis_index("c")
    sid = lax.axis_index("s")
    wid = sid * NC + cid

    def load_idx(f, _):
        pltpu.async_copy(idx_hbm.at[f, pl.ds(wid * BPW, BPW)],
                         idx_v.at[pl.ds(f * BPW, BPW)], sem)
        return _

    lax.fori_loop(0, N_FIELDS, load_idx, 0)

    base = sid * PART
    bufs = (stage_a, stage_b)
    pltpu.async_copy(tabm_hbm.at[pl.ds(base, CHT)], bufs[0], sem2)
    for k in range(8):
        b = bufs[k & 1]
        pltpu.make_async_copy(tabm_hbm.at[pl.ds(base + k * CHT, CHT)],
                              b, sem2).wait()
        if k < 7:
            pltpu.async_copy(tabm_hbm.at[pl.ds(base + (k + 1) * CHT, CHT)],
                             bufs[(k + 1) & 1], sem2)
        pltpu.sync_copy(b, tab_s.at[pl.ds(base + k * CHT, CHT)])

    @pl.when(sid == NS - 1)
    def _tail():
        pltpu.sync_copy(tabt_hbm, stage_a.at[pl.ds(0, TAB_TAIL)])
        pltpu.sync_copy(stage_a.at[pl.ds(0, TAB_TAIL)],
                        tab_s.at[pl.ds(TAB_MAIN, TAB_TAIL)])

    pltpu.make_async_copy(idx_hbm.at[0, pl.ds(0, N_FIELDS * BPW)],
                          idx_v, sem).wait()
    plsc.subcore_barrier()

    def fire(f, _):
        pltpu.async_copy(tab_s.at[idx_v.at[pl.ds(f * BPW, BPW)]],
                         vals_v.at[pl.ds(f * BPW, BPW)], sem)
        return _

    lax.fori_loop(0, N_FIELDS, fire, 0)

    pltpu.make_async_copy(tabm_hbm.at[pl.ds(0, N_FIELDS * BPW)],
                          vals_v, sem).wait()

    def reduce_chunk(j, _):
        col = j * L
        acc = vals_v[pl.ds(col, L)]
        for f in range(1, N_FIELDS):
            acc = acc + vals_v[pl.ds(f * BPW + col, L)]
        acc_v[pl.ds(col, L)] = acc
        return _

    lax.fori_loop(0, BPW // L, reduce_chunk, 0)

    pltpu.sync_copy(acc_v, out_hbm.at[pl.ds(wid * BPW, BPW)])


def kernel(row_sparse_feat, w):
    idx_t = row_sparse_feat.T
    tab_main = w[:TAB_MAIN].reshape(TAB_MAIN)
    tab_tail = jnp.pad(w[TAB_MAIN:],
                       ((0, TAB_TAIL - (FEAT_LEN - TAB_MAIN)), (0, 0))
                       ).reshape(TAB_TAIL)
    out = _lookup_sum(idx_t, tab_main, tab_tail)
    return out.reshape(BATCH, 1)


# speedup vs baseline: 3.7188x; 1.1389x over previous
import functools

import jax
import jax.numpy as jnp
from jax import lax
from jax.experimental import pallas as pl
from jax.experimental.pallas import tpu as pltpu
from jax.experimental.pallas import tpu_sc as plsc

FEAT_LEN = 1000000
BATCH = 16384
N_FIELDS = 26

NC = 2
NS = 16
L = 16
NW = NC * NS
BPW = BATCH // NW
CH = 128
NCH = BPW * N_FIELDS // CH

TAB_MAIN = 999424
TAB_TAIL = 1024
TAB_PAD = TAB_MAIN + TAB_TAIL
PART = TAB_MAIN // NS
CHT = PART // 8

_mesh = plsc.VectorSubcoreMesh(core_axis_name="c", subcore_axis_name="s")


@functools.partial(
    pl.kernel,
    out_type=jax.ShapeDtypeStruct((BATCH,), jnp.float32),
    mesh=_mesh,
    compiler_params=pltpu.CompilerParams(needs_layout_passes=False),
    scratch_types=[
        pltpu.VMEM((N_FIELDS * BPW,), jnp.int32),
        pltpu.VMEM((N_FIELDS * BPW,), jnp.float32),
        pltpu.VMEM((BPW,), jnp.float32),
        pltpu.VMEM_SHARED((TAB_PAD,), jnp.float32),
        pltpu.VMEM((CHT,), jnp.float32),
        pltpu.VMEM((CHT,), jnp.float32),
        pltpu.SemaphoreType.DMA,
        pltpu.SemaphoreType.DMA,
    ],
)
def _lookup_sum(idx_hbm, tab2_hbm, out_hbm, idx_v, vals_v, acc_v,
                tab_s, stage_a, stage_b, sem, sem2):
    cid = lax.axis_index("c")
    sid = lax.axis_index("s")
    wid = sid * NC + cid

    def load_idx(f, _):
        pltpu.async_copy(idx_hbm.at[f, pl.ds(wid * BPW, BPW)],
                         idx_v.at[pl.ds(f * BPW, BPW)], sem)
        return _

    lax.fori_loop(0, N_FIELDS, load_idx, 0)

    base = sid * PART
    bufs = (stage_a, stage_b)
    pltpu.async_copy(tab2_hbm.at[0, pl.ds(base, CHT)], bufs[0], sem2)
    for k in range(8):
        b = bufs[k & 1]
        pltpu.make_async_copy(tab2_hbm.at[0, pl.ds(base + k * CHT, CHT)],
                              b, sem2).wait()
        if k < 7:
            pltpu.async_copy(tab2_hbm.at[0, pl.ds(base + (k + 1) * CHT, CHT)],
                             bufs[(k + 1) & 1], sem2)
        pltpu.sync_copy(b, tab_s.at[pl.ds(base + k * CHT, CHT)])

    @pl.when(sid == NS - 1)
    def _tail():
        pltpu.sync_copy(tab2_hbm.at[0, pl.ds(TAB_MAIN, FEAT_LEN - TAB_MAIN)],
                        stage_a.at[pl.ds(0, FEAT_LEN - TAB_MAIN)])
        pltpu.sync_copy(stage_a.at[pl.ds(0, FEAT_LEN - TAB_MAIN)],
                        tab_s.at[pl.ds(TAB_MAIN, FEAT_LEN - TAB_MAIN)])

    pltpu.make_async_copy(idx_hbm.at[0, pl.ds(0, N_FIELDS * BPW)],
                          idx_v, sem).wait()
    plsc.subcore_barrier()

    def fire(f, _):
        pltpu.async_copy(tab_s.at[idx_v.at[pl.ds(f * BPW, BPW)]],
                         vals_v.at[pl.ds(f * BPW, BPW)], sem)
        return _

    lax.fori_loop(0, N_FIELDS, fire, 0)

    pltpu.make_async_copy(tab2_hbm.at[0, pl.ds(0, N_FIELDS * BPW)],
                          vals_v, sem).wait()

    def reduce_chunk(j, _):
        col = j * L
        acc = vals_v[pl.ds(col, L)]
        for f in range(1, N_FIELDS):
            acc = acc + vals_v[pl.ds(f * BPW + col, L)]
        acc_v[pl.ds(col, L)] = acc
        return _

    lax.fori_loop(0, BPW // L, reduce_chunk, 0)

    pltpu.sync_copy(acc_v, out_hbm.at[pl.ds(wid * BPW, BPW)])


def kernel(row_sparse_feat, w):
    out = _lookup_sum(row_sparse_feat.T, w.T)
    return out.reshape(BATCH, 1)
